# Initial kernel scaffold; baseline (speedup 1.0000x reference)
import jax
import jax.numpy as jnp
from jax.experimental import pallas as pl


def kernel(x, edge_index, edge_attr, batch, edge_W, edge_b, W1, b1, gamma, beta, W2, b2, Wo1, bo1, Wo2, bo2):
    raise NotImplementedError("write your pallas kernel here")


# speedup vs baseline: 2.5066x; 2.5066x over previous
import functools

import jax
import jax.numpy as jnp
from jax import lax
from jax.experimental import pallas as pl
from jax.experimental.pallas import tpu as pltpu
from jax.experimental.pallas import tpu_sc as plsc

_N = 10000
_E = 320000
_D = 128
_DE = 16
_L = 3
_G = 64

_NC = 2
_NS = 16
_NW = _NC * _NS
_EPT = _E // _NW
_CH = 80
_NCH = _EPT // _CH
_NP = 10240
_NPT = _NP // _NS
_ZR = 128

_BE = 3200


def _embed_body(attr_ref, w_ref, b_ref, out_ref):
    a = attr_ref[...]
    for l in range(_L):
        out_ref[l] = (
            jnp.dot(a, w_ref[l], preferred_element_type=jnp.float32) + b_ref[l]
        )


_embed = pl.pallas_call(
    _embed_body,
    grid=(_E // _BE,),
    in_specs=[
        pl.BlockSpec((_BE, _DE), lambda i: (i, 0)),
        pl.BlockSpec((_L, _DE, _D), lambda i: (0, 0, 0)),
        pl.BlockSpec((_L, 1, _D), lambda i: (0, 0, 0)),
    ],
    out_specs=pl.BlockSpec((_L, _BE, _D), lambda i: (0, i, 0)),
    out_shape=jax.ShapeDtypeStruct((_L, _E, _D), jnp.float32),
)


def _msgpass_body(l, h_hbm, ea_hbm, src_hbm, dst_hbm, out_hbm,
                  sidx, didx, msgb, hb, zb, acc):
    cid = lax.axis_index("c")
    sid = lax.axis_index("s")
    wid = cid * _NS + sid

    zero16 = jnp.zeros((16,), jnp.float32)

    def _zrow(i, c):
        for j in range(8):
            zb[i, pl.ds(j * 16, 16)] = zero16
        return c

    lax.fori_loop(0, _ZR, _zrow, 0)
    for t in range(_NPT // _ZR):
        pltpu.sync_copy(zb, acc.at[pl.ds(sid * _NPT + t * _ZR, _ZR)])
    plsc.subcore_barrier()

    gbase = l * _E + wid * _EPT

    def _chunk(k, c):
        pltpu.sync_copy(src_hbm.at[wid, k], sidx.at[0])
        pltpu.sync_copy(dst_hbm.at[wid, k], didx.at[0])
        pltpu.sync_copy(ea_hbm.at[pl.ds(gbase + k * _CH, _CH)], msgb)
        pltpu.sync_copy(h_hbm.at[sidx.at[0]], hb)

        def _relu(e, c2):
            for j in range(8):
                sl = pl.ds(j * 16, 16)
                msgb[e, sl] = jnp.maximum(msgb[e, sl] + hb[e, sl], 0.0)
            return c2

        lax.fori_loop(0, _CH, _relu, 0)
        pltpu.sync_copy(msgb, acc.at[didx.at[0]], add=True)
        return c

    lax.fori_loop(0, _NCH, _chunk, 0)
    plsc.subcore_barrier()

    for t in range(_NPT // _ZR):
        sl = pl.ds(sid * _NPT + t * _ZR, _ZR)
        pltpu.sync_copy(acc.at[sl], out_hbm.at[cid, sl])


@functools.cache
def _make_msgpass(l):
    return functools.partial(
        pl.kernel,
        mesh=plsc.VectorSubcoreMesh(core_axis_name="c", subcore_axis_name="s",
                                    num_cores=_NC, num_subcores=_NS),
        out_type=jax.ShapeDtypeStruct((_NC, _NP, _D), jnp.float32),
        scratch_types=[
            pltpu.VMEM((2, _CH), jnp.int32),
            pltpu.VMEM((2, _CH), jnp.int32),
            pltpu.VMEM((_CH, _D), jnp.float32),
            pltpu.VMEM((_CH, _D), jnp.float32),
            pltpu.VMEM((_ZR, _D), jnp.float32),
            pltpu.VMEM_SHARED((_NP, _D), jnp.float32),
        ],
    )(functools.partial(_msgpass_body, l))


def _mlp_body(h_ref, p_ref, w1_ref, b1_ref, g_ref, be_ref, w2_ref, b2_ref,
              out_ref):
    z = h_ref[...] + p_ref[0, :_N] + p_ref[1, :_N]
    z = jnp.dot(z, w1_ref[...], preferred_element_type=jnp.float32) + b1_ref[...]
    mu = jnp.mean(z, axis=0, keepdims=True)
    var = jnp.mean((z - mu) * (z - mu), axis=0, keepdims=True)
    z = (z - mu) / jnp.sqrt(var + 1e-5) * g_ref[...] + be_ref[...]
    z = jnp.maximum(z, 0.0)
    z = jnp.dot(z, w2_ref[...], preferred_element_type=jnp.float32) + b2_ref[...]
    out_ref[...] = jnp.maximum(z, 0.0)


_mlp = pl.pallas_call(
    _mlp_body,
    out_shape=jax.ShapeDtypeStruct((_N, _D), jnp.float32),
)


def _pool_body(h_ref, b_ref, wo1_ref, bo1_ref, wo2_ref, bo2_ref, out_ref):
    bids = b_ref[...]
    gi = lax.broadcasted_iota(jnp.int32, (_N, _G), 1)
    mask = (bids == gi).astype(jnp.float32)
    dn = (((0,), (0,)), ((), ()))
    sums = lax.dot_general(mask, h_ref[...], dn,
                           preferred_element_type=jnp.float32)
    ones = jnp.ones((_N, 1), jnp.float32)
    cnt = lax.dot_general(mask, ones, dn,
                          preferred_element_type=jnp.float32)
    pooled = sums / jnp.maximum(cnt, 1.0)
    t = jnp.maximum(
        jnp.dot(pooled, wo1_ref[...], preferred_element_type=jnp.float32)
        + bo1_ref[...], 0.0)
    out_ref[...] = (
        jnp.dot(t, wo2_ref[...], preferred_element_type=jnp.float32)
        + bo2_ref[...])


_pool = pl.pallas_call(
    _pool_body,
    out_shape=jax.ShapeDtypeStruct((_G, _D), jnp.float32),
)


def kernel(x, edge_index, edge_attr, batch, edge_W, edge_b, W1, b1, gamma,
           beta, W2, b2, Wo1, bo1, Wo2, bo2):
    src = edge_index[0].reshape(_NW, _NCH, _CH)
    dst = edge_index[1].reshape(_NW, _NCH, _CH)

    ea_all = _embed(edge_attr, edge_W, edge_b.reshape(_L, 1, _D))
    ea_flat = ea_all.reshape(_L * _E, _D)

    h = x
    for l in range(_L):
        parts = _make_msgpass(l)(h, ea_flat, src, dst)
        h = _mlp(h, parts, W1[l], b1[l].reshape(1, _D),
                 gamma[l].reshape(1, _D), beta[l].reshape(1, _D),
                 W2[l], b2[l].reshape(1, _D))

    return _pool(h, batch.reshape(_N, 1), Wo1, bo1.reshape(1, _D),
                 Wo2, bo2.reshape(1, _D))
